# Initial kernel scaffold; baseline (speedup 1.0000x reference)
import jax
import jax.numpy as jnp
from jax.experimental import pallas as pl


def kernel(x, edge_index, W1, b1, Wmu, bmu, Wls, bls):
    raise NotImplementedError("write your pallas kernel here")


# speedup vs baseline: 3.3137x; 3.3137x over previous
import functools

import jax
import jax.numpy as jnp
from jax.experimental import pallas as pl


def _dense_block(x_ref, w_ref, b_ref, o_ref):
    o_ref[...] = (
        jnp.dot(x_ref[...], w_ref[...], preferred_element_type=jnp.float32)
        + b_ref[...]
    )


def _matmul(x, W, b):
    n, k = x.shape
    m = W.shape[1]
    bn = 1000
    return pl.pallas_call(
        _dense_block,
        grid=(n // bn,),
        in_specs=[
            pl.BlockSpec((bn, k), lambda i: (i, 0)),
            pl.BlockSpec((k, m), lambda i: (0, 0)),
            pl.BlockSpec((1, m), lambda i: (0, 0)),
        ],
        out_specs=pl.BlockSpec((bn, m), lambda i: (i, 0)),
        out_shape=jax.ShapeDtypeStruct((n, m), jnp.float32),
    )(x, W, b.reshape(1, m))


def kernel(x, edge_index, W1, b1, Wmu, bmu, Wls, bls):
    n = x.shape[0]
    src, dst = edge_index[0], edge_index[1]
    deg = jax.ops.segment_sum(jnp.ones_like(src, jnp.float32), dst, num_segments=n) + 1.0
    dinv = jax.lax.rsqrt(deg)[:, None]

    u1 = x * dinv
    s1 = jax.ops.segment_sum(u1[src], dst, num_segments=n) + u1
    t1 = s1 * dinv

    h = jax.nn.relu(_matmul(t1, W1, b1))
    h = h / jnp.maximum(jnp.linalg.norm(h, axis=1, keepdims=True), 1e-12)

    u2 = h * dinv
    s2 = jax.ops.segment_sum(u2[src], dst, num_segments=n) + u2
    aggh = s2 * dinv

    mu = _matmul(aggh, Wmu, bmu)
    logstd = _matmul(aggh, Wls, bls)
    return (mu, logstd)


# speedup vs baseline: 17.8655x; 5.3913x over previous
import functools

import jax
import jax.numpy as jnp
from jax import lax
from jax.experimental import pallas as pl
from jax.experimental.pallas import tpu as pltpu
from jax.experimental.pallas import tpu_sc as plsc

N = 10000
E = 320000
NC = 2
NS = 16
CHUNK = 128
BLK = 16
ACC_ROWS = 10240
ROWS_PER_TILE = ACC_ROWS // NS


def _mesh():
    return plsc.VectorSubcoreMesh(core_axis_name="c", subcore_axis_name="s")


def _sc_hist(dsti, zeros128, ones_h, n_chunks):
    @functools.partial(
        pl.kernel,
        out_type=jax.ShapeDtypeStruct((NC * ACC_ROWS, 128), jnp.float32),
        mesh=_mesh(),
        scratch_types=[
            pltpu.VMEM_SHARED((ACC_ROWS, 128), jnp.float32),
            pltpu.VMEM((BLK, CHUNK), jnp.int32),
            pltpu.VMEM((CHUNK, 128), jnp.float32),
        ],
    )
    def k(dsti_hbm, zeros_hbm, ones_hbm, out_hbm, acc, dstv, ones):
        c = lax.axis_index("c")
        s = lax.axis_index("s")
        pltpu.sync_copy(ones_hbm, ones)
        pltpu.sync_copy(zeros_hbm, acc.at[pl.ds(s * ROWS_PER_TILE, ROWS_PER_TILE)])
        plsc.subcore_barrier()

        def blk_body(b, carry):
            pltpu.sync_copy(dsti_hbm.at[c, s, pl.ds(b * BLK, BLK)], dstv)

            def body(j, carry2):
                pltpu.sync_copy(ones, acc.at[dstv.at[j]], add=True)
                return carry2

            lax.fori_loop(0, BLK, body, 0)
            return carry

        lax.fori_loop(0, n_chunks // BLK, blk_body, 0)
        plsc.subcore_barrier()
        pltpu.sync_copy(acc.at[pl.ds(s * ROWS_PER_TILE, ROWS_PER_TILE)],
                        out_hbm.at[pl.ds(c * ACC_ROWS + s * ROWS_PER_TILE, ROWS_PER_TILE)])

    return k(dsti, zeros128, ones_h)


def _sc_agg(u, srci, dsti, zeros128, n_chunks):
    @functools.partial(
        pl.kernel,
        out_type=jax.ShapeDtypeStruct((NC * ACC_ROWS, 128), jnp.float32),
        mesh=_mesh(),
        scratch_types=[
            pltpu.VMEM_SHARED((ACC_ROWS, 128), jnp.float32),
            pltpu.VMEM((BLK, CHUNK), jnp.int32),
            pltpu.VMEM((BLK, CHUNK), jnp.int32),
            pltpu.VMEM((CHUNK, 128), jnp.float32),
            pltpu.SemaphoreType.DMA,
        ],
    )
    def k(u_hbm, srci_hbm, dsti_hbm, zeros_hbm, out_hbm, acc, srcv, dstv, buf, sem):
        c = lax.axis_index("c")
        s = lax.axis_index("s")
        pltpu.sync_copy(zeros_hbm, acc.at[pl.ds(s * ROWS_PER_TILE, ROWS_PER_TILE)])
        plsc.subcore_barrier()

        def blk_body(b, carry):
            pltpu.sync_copy(srci_hbm.at[c, s, pl.ds(b * BLK, BLK)], srcv)
            pltpu.sync_copy(dsti_hbm.at[c, s, pl.ds(b * BLK, BLK)], dstv)

            def body(j, carry2):
                pltpu.async_copy(u_hbm.at[srcv.at[j]], buf, sem).wait()
                pltpu.sync_copy(buf, acc.at[dstv.at[j]], add=True)
                return carry2

            lax.fori_loop(0, BLK, body, 0)
            return carry

        lax.fori_loop(0, n_chunks // BLK, blk_body, 0)
        plsc.subcore_barrier()
        pltpu.sync_copy(acc.at[pl.ds(s * ROWS_PER_TILE, ROWS_PER_TILE)],
                        out_hbm.at[pl.ds(c * ACC_ROWS + s * ROWS_PER_TILE, ROWS_PER_TILE)])

    return k(u, srci, dsti, zeros128)


_BN = 1000


def _deg_inv(h0_ref, h1_ref):
    deg = h0_ref[:, 0:1] + h1_ref[:, 0:1] + 1.0
    return lax.rsqrt(deg)


def _tc_u1_body(h0_ref, h1_ref, x_ref, o_ref):
    o_ref[...] = x_ref[...] * _deg_inv(h0_ref, h1_ref)


def _tc_u1(h0, h1, x):
    nb = N // _BN
    return pl.pallas_call(
        _tc_u1_body,
        grid=(NC, nb),
        in_specs=[
            pl.BlockSpec((_BN, 16), lambda c, i: (i, 0)),
            pl.BlockSpec((_BN, 16), lambda c, i: (i, 0)),
            pl.BlockSpec((_BN, 128), lambda c, i: (i, 0)),
        ],
        out_specs=pl.BlockSpec((_BN, 128), lambda c, i: (c * (N // _BN) + i, 0)),
        out_shape=jax.ShapeDtypeStruct((NC * N, 128), jnp.float32),
    )(h0, h1, x)


def _tc_hidden_body(p0_ref, p1_ref, x_ref, h0_ref, h1_ref, w_ref, b_ref, o_ref):
    dinv = _deg_inv(h0_ref, h1_ref)
    u1 = x_ref[...] * dinv
    t1 = (p0_ref[...] + p1_ref[...] + u1) * dinv
    h = jnp.dot(t1, w_ref[...], preferred_element_type=jnp.float32) + b_ref[...]
    h = jnp.maximum(h, 0.0)
    nrm = jnp.sqrt(jnp.sum(h * h, axis=1, keepdims=True))
    h = h / jnp.maximum(nrm, 1e-12)
    o_ref[...] = h * dinv


def _tc_hidden(p0, p1, x, h0, h1, W1, b1):
    nb = N // _BN
    return pl.pallas_call(
        _tc_hidden_body,
        grid=(nb,),
        in_specs=[
            pl.BlockSpec((_BN, 128), lambda i: (i, 0)),
            pl.BlockSpec((_BN, 128), lambda i: (i, 0)),
            pl.BlockSpec((_BN, 128), lambda i: (i, 0)),
            pl.BlockSpec((_BN, 16), lambda i: (i, 0)),
            pl.BlockSpec((_BN, 16), lambda i: (i, 0)),
            pl.BlockSpec((128, 256), lambda i: (0, 0)),
            pl.BlockSpec((1, 256), lambda i: (0, 0)),
        ],
        out_specs=pl.BlockSpec((_BN, 256), lambda i: (i, 0)),
        out_shape=jax.ShapeDtypeStruct((N, 256), jnp.float32),
    )(p0, p1, x, h0, h1, W1, b1.reshape(1, 256))


def _tc_out_body(s0_ref, s1_ref, u0_ref, u1r_ref, h0_ref, h1_ref, w_ref, b_ref, o_ref):
    dinv = _deg_inv(h0_ref, h1_ref)
    a_lo = (s0_ref[...] + u0_ref[...]) * dinv
    a_hi = (s1_ref[...] + u1r_ref[...]) * dinv
    o_ref[...] = (
        jnp.dot(a_lo, w_ref[0:128, :], preferred_element_type=jnp.float32)
        + jnp.dot(a_hi, w_ref[128:256, :], preferred_element_type=jnp.float32)
        + b_ref[...]
    )


def _tc_out(s0, s1, u0, u1r, h0, h1, Wc, bc):
    nb = N // _BN
    return pl.pallas_call(
        _tc_out_body,
        grid=(nb,),
        in_specs=[
            pl.BlockSpec((_BN, 128), lambda i: (i, 0)),
            pl.BlockSpec((_BN, 128), lambda i: (i, 0)),
            pl.BlockSpec((_BN, 128), lambda i: (i, 0)),
            pl.BlockSpec((_BN, 128), lambda i: (i, 0)),
            pl.BlockSpec((_BN, 16), lambda i: (i, 0)),
            pl.BlockSpec((_BN, 16), lambda i: (i, 0)),
            pl.BlockSpec((256, 256), lambda i: (0, 0)),
            pl.BlockSpec((1, 256), lambda i: (0, 0)),
        ],
        out_specs=pl.BlockSpec((_BN, 256), lambda i: (i, 0)),
        out_shape=jax.ShapeDtypeStruct((N, 256), jnp.float32),
    )(s0, s1, u0, u1r, h0, h1, Wc, bc)


def _pad_chunks(arr, pad_vals):
    nc, ns, per_tile = arr.shape
    grain = BLK * CHUNK
    padded = -(-per_tile // grain) * grain
    pad = padded - per_tile
    if pad:
        padb = jnp.broadcast_to(pad_vals[..., :pad], (nc, ns, pad))
        arr = jnp.concatenate([arr, padb], axis=2)
    return arr.reshape(nc, ns, padded // CHUNK, CHUNK)


def kernel(x, edge_index, W1, b1, Wmu, bmu, Wls, bls):
    src = edge_index[0].astype(jnp.int32)
    dst = edge_index[1].astype(jnp.int32)
    coff = (jnp.arange(NC, dtype=jnp.int32) * N)[:, None, None]
    maxpad = BLK * CHUNK
    spread = (jnp.arange(maxpad, dtype=jnp.int32) % 64)[None, None, :]
    dpad = (N + jnp.arange(maxpad, dtype=jnp.int32) % 16)[None, None, :]

    per1 = E // (NC * NS)
    src1 = _pad_chunks(src.reshape(NC, NS, per1) + coff, coff + spread)
    dst1 = _pad_chunks(dst.reshape(NC, NS, per1), dpad)
    c1 = src1.shape[2]
    per2 = E // NS
    src2 = _pad_chunks(
        jnp.broadcast_to(src.reshape(1, NS, per2), (NC, NS, per2)) + coff,
        coff + spread)
    dst2 = _pad_chunks(jnp.broadcast_to(dst.reshape(1, NS, per2), (NC, NS, per2)), dpad)
    c2 = src2.shape[2]

    zeros128 = jnp.zeros((ROWS_PER_TILE, 128), jnp.float32)

    ones_h = jnp.ones((CHUNK, 128), jnp.float32)
    hist = _sc_hist(dst1, zeros128, ones_h, c1)
    h0, h1 = hist[:N, :16], hist[ACC_ROWS:ACC_ROWS + N, :16]

    u1d = _tc_u1(h0, h1, x)
    s1 = _sc_agg(u1d, src1, dst1, zeros128, c1)

    u2 = _tc_hidden(s1[:N], s1[ACC_ROWS:ACC_ROWS + N], x, h0, h1, W1, b1)
    u2f = jnp.concatenate([u2[:, :128], u2[:, 128:]], axis=0)
    s2 = _sc_agg(u2f, src2, dst2, zeros128, c2)

    Wc = jnp.concatenate([Wmu, Wls], axis=1)
    bc = jnp.concatenate([bmu, bls]).reshape(1, 256)
    out = _tc_out(s2[:N], s2[ACC_ROWS:ACC_ROWS + N], u2[:, :128], u2[:, 128:], h0, h1, Wc, bc)
    return (out[:, :128], out[:, 128:])


# speedup vs baseline: 21.6280x; 1.2106x over previous
import functools

import jax
import jax.numpy as jnp
from jax import lax
from jax.experimental import pallas as pl
from jax.experimental.pallas import tpu as pltpu
from jax.experimental.pallas import tpu_sc as plsc

N = 10000
E = 320000
NC = 2
NS = 16
CHUNK = 128
BLK = 16
ACC_ROWS = 10240
ROWS_PER_TILE = ACC_ROWS // NS


def _mesh():
    return plsc.VectorSubcoreMesh(core_axis_name="c", subcore_axis_name="s")


def _sc_hist(dsti, zeros128, ones_h, n_chunks):
    @functools.partial(
        pl.kernel,
        out_type=jax.ShapeDtypeStruct((NC * ACC_ROWS, 128), jnp.float32),
        mesh=_mesh(),
        scratch_types=[
            pltpu.VMEM_SHARED((ACC_ROWS, 128), jnp.float32),
            pltpu.VMEM((BLK, CHUNK), jnp.int32),
            pltpu.VMEM((CHUNK, 128), jnp.float32),
        ],
    )
    def k(dsti_hbm, zeros_hbm, ones_hbm, out_hbm, acc, dstv, ones):
        c = lax.axis_index("c")
        s = lax.axis_index("s")
        pltpu.sync_copy(ones_hbm, ones)
        pltpu.sync_copy(zeros_hbm, acc.at[pl.ds(s * ROWS_PER_TILE, ROWS_PER_TILE)])
        plsc.subcore_barrier()

        def blk_body(b, carry):
            pltpu.sync_copy(dsti_hbm.at[c, s, pl.ds(b * BLK, BLK)], dstv)

            def body(j, carry2):
                pltpu.sync_copy(ones, acc.at[dstv.at[j]], add=True)
                return carry2

            lax.fori_loop(0, BLK, body, 0)
            return carry

        lax.fori_loop(0, n_chunks // BLK, blk_body, 0)
        plsc.subcore_barrier()
        pltpu.sync_copy(acc.at[pl.ds(s * ROWS_PER_TILE, ROWS_PER_TILE)],
                        out_hbm.at[pl.ds(c * ACC_ROWS + s * ROWS_PER_TILE, ROWS_PER_TILE)])

    return k(dsti, zeros128, ones_h)


def _sc_agg(u, srci, dsti, zeros128, n_chunks):
    @functools.partial(
        pl.kernel,
        out_type=jax.ShapeDtypeStruct((NC * ACC_ROWS, 128), jnp.float32),
        mesh=_mesh(),
        scratch_types=[
            pltpu.VMEM_SHARED((ACC_ROWS, 128), jnp.float32),
            pltpu.VMEM((BLK, CHUNK), jnp.int32),
            pltpu.VMEM((BLK, CHUNK), jnp.int32),
            pltpu.VMEM((CHUNK, 128), jnp.float32),
            pltpu.VMEM((CHUNK, 128), jnp.float32),
            pltpu.SemaphoreType.DMA,
            pltpu.SemaphoreType.DMA,
        ],
    )
    def k(u_hbm, srci_hbm, dsti_hbm, zeros_hbm, out_hbm, acc,
          srcv, dstv, bufa, bufb, sema, semb):
        c = lax.axis_index("c")
        s = lax.axis_index("s")
        pltpu.sync_copy(zeros_hbm, acc.at[pl.ds(s * ROWS_PER_TILE, ROWS_PER_TILE)])
        plsc.subcore_barrier()

        def blk_body(b, carry):
            pltpu.sync_copy(srci_hbm.at[c, s, pl.ds(b * BLK, BLK)], srcv)
            pltpu.sync_copy(dsti_hbm.at[c, s, pl.ds(b * BLK, BLK)], dstv)
            pltpu.async_copy(u_hbm.at[srcv.at[0]], bufa, sema)

            def pair_body(p, carry2):
                j = p * 2
                pltpu.make_async_copy(u_hbm.at[srcv.at[j]], bufa, sema).wait()
                pltpu.async_copy(u_hbm.at[srcv.at[j + 1]], bufb, semb)
                pltpu.sync_copy(bufa, acc.at[dstv.at[j]], add=True)
                pltpu.make_async_copy(u_hbm.at[srcv.at[j + 1]], bufb, semb).wait()

                @pl.when(p + 1 < BLK // 2)
                def _():
                    pltpu.async_copy(u_hbm.at[srcv.at[j + 2]], bufa, sema)

                pltpu.sync_copy(bufb, acc.at[dstv.at[j + 1]], add=True)
                return carry2

            lax.fori_loop(0, BLK // 2, pair_body, 0)
            return carry

        lax.fori_loop(0, n_chunks // BLK, blk_body, 0)
        plsc.subcore_barrier()
        pltpu.sync_copy(acc.at[pl.ds(s * ROWS_PER_TILE, ROWS_PER_TILE)],
                        out_hbm.at[pl.ds(c * ACC_ROWS + s * ROWS_PER_TILE, ROWS_PER_TILE)])

    return k(u, srci, dsti, zeros128)


_BN = 1000


def _deg_inv(h0_ref, h1_ref):
    deg = h0_ref[:, 0:1] + h1_ref[:, 0:1] + 1.0
    return lax.rsqrt(deg)


def _tc_u1_body(h0_ref, h1_ref, x_ref, o_ref):
    o_ref[...] = x_ref[...] * _deg_inv(h0_ref, h1_ref)


def _tc_u1(h0, h1, x):
    nb = N // _BN
    return pl.pallas_call(
        _tc_u1_body,
        grid=(NC, nb),
        in_specs=[
            pl.BlockSpec((_BN, 16), lambda c, i: (i, 0)),
            pl.BlockSpec((_BN, 16), lambda c, i: (i, 0)),
            pl.BlockSpec((_BN, 128), lambda c, i: (i, 0)),
        ],
        out_specs=pl.BlockSpec((_BN, 128), lambda c, i: (c * (N // _BN) + i, 0)),
        out_shape=jax.ShapeDtypeStruct((NC * N, 128), jnp.float32),
    )(h0, h1, x)


def _tc_hidden_body(p0_ref, p1_ref, x_ref, h0_ref, h1_ref, w_ref, b_ref, o_ref):
    dinv = _deg_inv(h0_ref, h1_ref)
    u1 = x_ref[...] * dinv
    t1 = (p0_ref[...] + p1_ref[...] + u1) * dinv
    h = jnp.dot(t1, w_ref[...], preferred_element_type=jnp.float32) + b_ref[...]
    h = jnp.maximum(h, 0.0)
    nrm = jnp.sqrt(jnp.sum(h * h, axis=1, keepdims=True))
    h = h / jnp.maximum(nrm, 1e-12)
    o_ref[...] = h * dinv


def _tc_hidden(p0, p1, x, h0, h1, W1, b1):
    nb = N // _BN
    return pl.pallas_call(
        _tc_hidden_body,
        grid=(nb,),
        in_specs=[
            pl.BlockSpec((_BN, 128), lambda i: (i, 0)),
            pl.BlockSpec((_BN, 128), lambda i: (i, 0)),
            pl.BlockSpec((_BN, 128), lambda i: (i, 0)),
            pl.BlockSpec((_BN, 16), lambda i: (i, 0)),
            pl.BlockSpec((_BN, 16), lambda i: (i, 0)),
            pl.BlockSpec((128, 256), lambda i: (0, 0)),
            pl.BlockSpec((1, 256), lambda i: (0, 0)),
        ],
        out_specs=pl.BlockSpec((_BN, 256), lambda i: (i, 0)),
        out_shape=jax.ShapeDtypeStruct((N, 256), jnp.float32),
    )(p0, p1, x, h0, h1, W1, b1.reshape(1, 256))


def _tc_out_body(s0_ref, s1_ref, u0_ref, u1r_ref, h0_ref, h1_ref, w_ref, b_ref, o_ref):
    dinv = _deg_inv(h0_ref, h1_ref)
    a_lo = (s0_ref[...] + u0_ref[...]) * dinv
    a_hi = (s1_ref[...] + u1r_ref[...]) * dinv
    o_ref[...] = (
        jnp.dot(a_lo, w_ref[0:128, :], preferred_element_type=jnp.float32)
        + jnp.dot(a_hi, w_ref[128:256, :], preferred_element_type=jnp.float32)
        + b_ref[...]
    )


def _tc_out(s0, s1, u0, u1r, h0, h1, Wc, bc):
    nb = N // _BN
    return pl.pallas_call(
        _tc_out_body,
        grid=(nb,),
        in_specs=[
            pl.BlockSpec((_BN, 128), lambda i: (i, 0)),
            pl.BlockSpec((_BN, 128), lambda i: (i, 0)),
            pl.BlockSpec((_BN, 128), lambda i: (i, 0)),
            pl.BlockSpec((_BN, 128), lambda i: (i, 0)),
            pl.BlockSpec((_BN, 16), lambda i: (i, 0)),
            pl.BlockSpec((_BN, 16), lambda i: (i, 0)),
            pl.BlockSpec((256, 256), lambda i: (0, 0)),
            pl.BlockSpec((1, 256), lambda i: (0, 0)),
        ],
        out_specs=pl.BlockSpec((_BN, 256), lambda i: (i, 0)),
        out_shape=jax.ShapeDtypeStruct((N, 256), jnp.float32),
    )(s0, s1, u0, u1r, h0, h1, Wc, bc)


def _pad_chunks(arr, pad_vals):
    nc, ns, per_tile = arr.shape
    grain = BLK * CHUNK
    padded = -(-per_tile // grain) * grain
    pad = padded - per_tile
    if pad:
        padb = jnp.broadcast_to(pad_vals[..., :pad], (nc, ns, pad))
        arr = jnp.concatenate([arr, padb], axis=2)
    return arr.reshape(nc, ns, padded // CHUNK, CHUNK)


def kernel(x, edge_index, W1, b1, Wmu, bmu, Wls, bls):
    src = edge_index[0].astype(jnp.int32)
    dst = edge_index[1].astype(jnp.int32)
    coff = (jnp.arange(NC, dtype=jnp.int32) * N)[:, None, None]
    maxpad = BLK * CHUNK
    spread = (jnp.arange(maxpad, dtype=jnp.int32) % 64)[None, None, :]
    dpad = (N + jnp.arange(maxpad, dtype=jnp.int32) % 16)[None, None, :]

    per1 = E // (NC * NS)
    src1 = _pad_chunks(src.reshape(NC, NS, per1) + coff, coff + spread)
    dst1 = _pad_chunks(dst.reshape(NC, NS, per1), dpad)
    c1 = src1.shape[2]
    per2 = E // NS
    src2 = _pad_chunks(
        jnp.broadcast_to(src.reshape(1, NS, per2), (NC, NS, per2)) + coff,
        coff + spread)
    dst2 = _pad_chunks(jnp.broadcast_to(dst.reshape(1, NS, per2), (NC, NS, per2)), dpad)
    c2 = src2.shape[2]

    zeros128 = jnp.zeros((ROWS_PER_TILE, 128), jnp.float32)

    ones_h = jnp.ones((CHUNK, 128), jnp.float32)
    hist = _sc_hist(dst1, zeros128, ones_h, c1)
    h0, h1 = hist[:N, :16], hist[ACC_ROWS:ACC_ROWS + N, :16]

    u1d = _tc_u1(h0, h1, x)
    s1 = _sc_agg(u1d, src1, dst1, zeros128, c1)

    u2 = _tc_hidden(s1[:N], s1[ACC_ROWS:ACC_ROWS + N], x, h0, h1, W1, b1)
    u2f = jnp.concatenate([u2[:, :128], u2[:, 128:]], axis=0)
    s2 = _sc_agg(u2f, src2, dst2, zeros128, c2)

    Wc = jnp.concatenate([Wmu, Wls], axis=1)
    bc = jnp.concatenate([bmu, bls]).reshape(1, 256)
    out = _tc_out(s2[:N], s2[ACC_ROWS:ACC_ROWS + N], u2[:, :128], u2[:, 128:], h0, h1, Wc, bc)
    return (out[:, :128], out[:, 128:])


# speedup vs baseline: 23.6608x; 1.0940x over previous
import functools

import jax
import jax.numpy as jnp
from jax import lax
from jax.experimental import pallas as pl
from jax.experimental.pallas import tpu as pltpu
from jax.experimental.pallas import tpu_sc as plsc

N = 10000
E = 320000
NC = 2
NS = 16
CHUNK = 128
BLK = 16
ACC_ROWS = 10240
ROWS_PER_TILE = ACC_ROWS // NS


def _mesh():
    return plsc.VectorSubcoreMesh(core_axis_name="c", subcore_axis_name="s")


def _sc_hist(dsti, n_chunks):
    prt8 = ROWS_PER_TILE // 8
    @functools.partial(
        pl.kernel,
        out_type=jax.ShapeDtypeStruct((NC * ACC_ROWS // 8, 128), jnp.float32),
        mesh=_mesh(),
        scratch_types=[
            pltpu.VMEM_SHARED((ACC_ROWS, 16), jnp.float32),
            pltpu.VMEM((BLK, CHUNK), jnp.int32),
            pltpu.VMEM((CHUNK, 16), jnp.float32),
            pltpu.VMEM((ROWS_PER_TILE, 16), jnp.float32),
            pltpu.VMEM((prt8, 128), jnp.float32),
        ],
    )
    def k(dsti_hbm, out_hbm, acc, dstv, ones, tmp, packed):
        c = lax.axis_index("c")
        s = lax.axis_index("s")
        one = jnp.full((16,), 1.0, dtype=jnp.float32)
        zero = jnp.zeros((16,), dtype=jnp.float32)
        for i in range(CHUNK):
            ones[i] = one

        def zero_body(i, carry):
            tmp[i] = zero
            return carry

        lax.fori_loop(0, ROWS_PER_TILE, zero_body, 0)
        pltpu.sync_copy(tmp, acc.at[pl.ds(s * ROWS_PER_TILE, ROWS_PER_TILE)])
        plsc.subcore_barrier()

        def blk_body(b, carry):
            pltpu.sync_copy(dsti_hbm.at[c, s, pl.ds(b * BLK, BLK)], dstv)

            def body(j, carry2):
                pltpu.sync_copy(ones, acc.at[dstv.at[j]], add=True)
                return carry2

            lax.fori_loop(0, BLK, body, 0)
            return carry

        lax.fori_loop(0, n_chunks // BLK, blk_body, 0)
        plsc.subcore_barrier()
        pltpu.sync_copy(acc.at[pl.ds(s * ROWS_PER_TILE, ROWS_PER_TILE)], tmp)

        def repack_body(i, carry):
            packed[i // 8, pl.ds((i % 8) * 16, 16)] = tmp[i]
            return carry

        lax.fori_loop(0, ROWS_PER_TILE, repack_body, 0)
        pltpu.sync_copy(packed, out_hbm.at[pl.ds(c * (ACC_ROWS // 8) + s * prt8, prt8)])

    return k(dsti)


def _sc_agg(u, srci, dsti, zeros128, n_chunks):
    @functools.partial(
        pl.kernel,
        out_type=jax.ShapeDtypeStruct((NC * ACC_ROWS, 128), jnp.float32),
        mesh=_mesh(),
        scratch_types=[
            pltpu.VMEM_SHARED((ACC_ROWS, 128), jnp.float32),
            pltpu.VMEM((BLK, CHUNK), jnp.int32),
            pltpu.VMEM((BLK, CHUNK), jnp.int32),
            pltpu.VMEM((CHUNK, 128), jnp.float32),
            pltpu.VMEM((CHUNK, 128), jnp.float32),
            pltpu.SemaphoreType.DMA,
            pltpu.SemaphoreType.DMA,
        ],
    )
    def k(u_hbm, srci_hbm, dsti_hbm, zeros_hbm, out_hbm, acc,
          srcv, dstv, bufa, bufb, sema, semb):
        c = lax.axis_index("c")
        s = lax.axis_index("s")
        pltpu.sync_copy(zeros_hbm, acc.at[pl.ds(s * ROWS_PER_TILE, ROWS_PER_TILE)])
        plsc.subcore_barrier()

        def blk_body(b, carry):
            pltpu.sync_copy(srci_hbm.at[c, s, pl.ds(b * BLK, BLK)], srcv)
            pltpu.sync_copy(dsti_hbm.at[c, s, pl.ds(b * BLK, BLK)], dstv)
            pltpu.async_copy(u_hbm.at[srcv.at[0]], bufa, sema)

            def pair_body(p, carry2):
                j = p * 2
                pltpu.make_async_copy(u_hbm.at[srcv.at[j]], bufa, sema).wait()
                pltpu.async_copy(u_hbm.at[srcv.at[j + 1]], bufb, semb)
                pltpu.sync_copy(bufa, acc.at[dstv.at[j]], add=True)
                pltpu.make_async_copy(u_hbm.at[srcv.at[j + 1]], bufb, semb).wait()

                @pl.when(p + 1 < BLK // 2)
                def _():
                    pltpu.async_copy(u_hbm.at[srcv.at[j + 2]], bufa, sema)

                pltpu.sync_copy(bufb, acc.at[dstv.at[j + 1]], add=True)
                return carry2

            lax.fori_loop(0, BLK // 2, pair_body, 0)
            return carry

        lax.fori_loop(0, n_chunks // BLK, blk_body, 0)
        plsc.subcore_barrier()
        pltpu.sync_copy(acc.at[pl.ds(s * ROWS_PER_TILE, ROWS_PER_TILE)],
                        out_hbm.at[pl.ds(c * ACC_ROWS + s * ROWS_PER_TILE, ROWS_PER_TILE)])

    return k(u, srci, dsti, zeros128)


_BN = 1000


def _deg_inv(h0_ref, h1_ref):
    deg = h0_ref[:, 0:1] + h1_ref[:, 0:1] + 1.0
    return lax.rsqrt(deg)


def _tc_u1_body(h0_ref, h1_ref, x_ref, o_ref):
    o_ref[...] = x_ref[...] * _deg_inv(h0_ref, h1_ref)


def _tc_u1(h0, h1, x):
    nb = N // _BN
    return pl.pallas_call(
        _tc_u1_body,
        grid=(nb,),
        in_specs=[
            pl.BlockSpec((_BN, 16), lambda i: (i, 0)),
            pl.BlockSpec((_BN, 16), lambda i: (i, 0)),
            pl.BlockSpec((_BN, 128), lambda i: (i, 0)),
        ],
        out_specs=pl.BlockSpec((_BN, 128), lambda i: (i, 0)),
        out_shape=jax.ShapeDtypeStruct((N, 128), jnp.float32),
    )(h0, h1, x)


def _tc_hidden_body(p0_ref, p1_ref, x_ref, h0_ref, h1_ref, w_ref, b_ref, o_ref):
    dinv = _deg_inv(h0_ref, h1_ref)
    u1 = x_ref[...] * dinv
    t1 = (p0_ref[...] + p1_ref[...] + u1) * dinv
    h = jnp.dot(t1, w_ref[...], preferred_element_type=jnp.float32) + b_ref[...]
    h = jnp.maximum(h, 0.0)
    nrm = jnp.sqrt(jnp.sum(h * h, axis=1, keepdims=True))
    h = h / jnp.maximum(nrm, 1e-12)
    o_ref[...] = h * dinv


def _tc_hidden(p0, p1, x, h0, h1, W1, b1):
    nb = N // _BN
    return pl.pallas_call(
        _tc_hidden_body,
        grid=(nb,),
        in_specs=[
            pl.BlockSpec((_BN, 128), lambda i: (i, 0)),
            pl.BlockSpec((_BN, 128), lambda i: (i, 0)),
            pl.BlockSpec((_BN, 128), lambda i: (i, 0)),
            pl.BlockSpec((_BN, 16), lambda i: (i, 0)),
            pl.BlockSpec((_BN, 16), lambda i: (i, 0)),
            pl.BlockSpec((128, 256), lambda i: (0, 0)),
            pl.BlockSpec((1, 256), lambda i: (0, 0)),
        ],
        out_specs=pl.BlockSpec((_BN, 256), lambda i: (i, 0)),
        out_shape=jax.ShapeDtypeStruct((N, 256), jnp.float32),
    )(p0, p1, x, h0, h1, W1, b1.reshape(1, 256))


def _tc_out_body(s0_ref, s1_ref, u0_ref, u1r_ref, h0_ref, h1_ref, w_ref, b_ref, o_ref):
    dinv = _deg_inv(h0_ref, h1_ref)
    a_lo = (s0_ref[...] + u0_ref[...]) * dinv
    a_hi = (s1_ref[...] + u1r_ref[...]) * dinv
    o_ref[...] = (
        jnp.dot(a_lo, w_ref[0:128, :], preferred_element_type=jnp.float32)
        + jnp.dot(a_hi, w_ref[128:256, :], preferred_element_type=jnp.float32)
        + b_ref[...]
    )


def _tc_out(s0, s1, u0, u1r, h0, h1, Wc, bc):
    nb = N // _BN
    return pl.pallas_call(
        _tc_out_body,
        grid=(nb,),
        in_specs=[
            pl.BlockSpec((_BN, 128), lambda i: (i, 0)),
            pl.BlockSpec((_BN, 128), lambda i: (i, 0)),
            pl.BlockSpec((_BN, 128), lambda i: (i, 0)),
            pl.BlockSpec((_BN, 128), lambda i: (i, 0)),
            pl.BlockSpec((_BN, 16), lambda i: (i, 0)),
            pl.BlockSpec((_BN, 16), lambda i: (i, 0)),
            pl.BlockSpec((256, 256), lambda i: (0, 0)),
            pl.BlockSpec((1, 256), lambda i: (0, 0)),
        ],
        out_specs=pl.BlockSpec((_BN, 256), lambda i: (i, 0)),
        out_shape=jax.ShapeDtypeStruct((N, 256), jnp.float32),
    )(s0, s1, u0, u1r, h0, h1, Wc, bc)


def _pad_chunks(arr, pad_vals):
    nc, ns, per_tile = arr.shape
    grain = BLK * CHUNK
    padded = -(-per_tile // grain) * grain
    pad = padded - per_tile
    if pad:
        padb = jnp.broadcast_to(pad_vals[..., :pad], (nc, ns, pad))
        arr = jnp.concatenate([arr, padb], axis=2)
    return arr.reshape(nc, ns, padded // CHUNK, CHUNK)


def kernel(x, edge_index, W1, b1, Wmu, bmu, Wls, bls):
    src = edge_index[0].astype(jnp.int32)
    dst = edge_index[1].astype(jnp.int32)
    coff = (jnp.arange(NC, dtype=jnp.int32) * N)[:, None, None]
    maxpad = BLK * CHUNK
    spread = (jnp.arange(maxpad, dtype=jnp.int32) % 64)[None, None, :]
    dpad = (N + jnp.arange(maxpad, dtype=jnp.int32) % 16)[None, None, :]

    per1 = E // (NC * NS)
    src1 = _pad_chunks(src.reshape(NC, NS, per1), jnp.broadcast_to(spread, (NC, 1, maxpad)))
    dst1 = _pad_chunks(dst.reshape(NC, NS, per1), dpad)
    c1 = src1.shape[2]
    per2 = E // NS
    src2 = _pad_chunks(
        jnp.broadcast_to(src.reshape(1, NS, per2), (NC, NS, per2)) + coff,
        coff + spread)
    dst2 = _pad_chunks(jnp.broadcast_to(dst.reshape(1, NS, per2), (NC, NS, per2)), dpad)
    c2 = src2.shape[2]

    zeros128 = jnp.zeros((ROWS_PER_TILE, 128), jnp.float32)

    hist = _sc_hist(dst1, c1).reshape(NC, ACC_ROWS, 16)
    h0, h1 = hist[0, :N], hist[1, :N]

    u1d = _tc_u1(h0, h1, x)
    s1 = _sc_agg(u1d, src1, dst1, zeros128, c1)

    u2 = _tc_hidden(s1[:N], s1[ACC_ROWS:ACC_ROWS + N], x, h0, h1, W1, b1)
    u2f = jnp.concatenate([u2[:, :128], u2[:, 128:]], axis=0)
    s2 = _sc_agg(u2f, src2, dst2, zeros128, c2)

    Wc = jnp.concatenate([Wmu, Wls], axis=1)
    bc = jnp.concatenate([bmu, bls]).reshape(1, 256)
    out = _tc_out(s2[:N], s2[ACC_ROWS:ACC_ROWS + N], u2[:, :128], u2[:, 128:], h0, h1, Wc, bc)
    return (out[:, :128], out[:, 128:])
